# baseline (speedup 1.0000x reference)
import functools

import jax
import jax.numpy as jnp
from jax import lax
from jax.experimental import pallas as pl
from jax.experimental.pallas import tpu as pltpu
from jax.experimental.pallas import tpu_sc as plsc


@functools.lru_cache(maxsize=None)
def _make_transpose(v: int, d: int):
    info = plsc.get_sparse_core_info()
    nw = info.num_cores * info.num_subcores
    full_blocks = v // 128
    tail = v % 128
    g = 4
    cols = g * 128
    batches = full_blocks // g
    base_batches = batches // nw
    extra_cols = full_blocks - base_batches * nw * g
    assert extra_cols <= nw
    mesh = plsc.VectorSubcoreMesh(core_axis_name="c", subcore_axis_name="s")

    scratch = (
        [pltpu.VMEM((d, cols), jnp.float32) for _ in range(2)]
        + [pltpu.VMEM((cols * d,), jnp.float32) for _ in range(2)]
        + [pltpu.SemaphoreType.DMA] * 4
    )

    @functools.partial(
        pl.kernel,
        mesh=mesh,
        out_type=jax.ShapeDtypeStruct((v * d,), jnp.float32),
        scratch_types=scratch,
        compiler_params=pltpu.CompilerParams(
            use_tc_tiling_on_sc=True, needs_layout_passes=False
        ),
    )
    def transpose_kernel(embt_hbm, tail_hbm, r_hbm,
                         i0, i1, o0, o1, si0, si1, so0, so1):
        in_v = (i0, i1)
        out_v = (o0, o1)
        sem_i = (si0, si1)
        sem_o = (so0, so1)
        wid = lax.axis_index("s") * info.num_cores + lax.axis_index("c")
        iota = lax.iota(jnp.int32, 16)
        iota_hi = iota + 16

        def start_in(c0, b, width):
            for jb in range(d // 8):
                pltpu.async_copy(
                    embt_hbm.at[pl.ds(jb * 8, 8), pl.ds(c0, width)],
                    in_v[b].at[pl.ds(jb * 8, 8), pl.ds(0, width)],
                    sem_i[b],
                )

        def wait_in(b, width):
            for jb in range(d // 8):
                pltpu.make_async_copy(
                    embt_hbm.at[pl.ds(jb * 8, 8), pl.ds(0, width)],
                    in_v[b].at[pl.ds(jb * 8, 8), pl.ds(0, width)],
                    sem_i[b],
                ).wait()

        def start_out(c0, b, width):
            pltpu.async_copy(
                out_v[b].at[pl.ds(0, width * d)],
                r_hbm.at[pl.ds(c0 * d, width * d)],
                sem_o[b],
            )

        def wait_out(b, width):
            pltpu.make_async_copy(
                out_v[b].at[pl.ds(0, width * d)],
                r_hbm.at[pl.ds(0, width * d)],
                sem_o[b],
            ).wait()

        iota_d = iota * d

        def transpose(b, width):
            def body(q, carry):
                base0 = q * 16 * d
                for j in range(d):
                    vals = in_v[b][j, pl.ds(q * 16, 16)]
                    plsc.store_scatter(out_v[b], [iota_d + (base0 + j)], vals)
                return carry

            lax.fori_loop(0, width // 16, body, 0, unroll=4)

        start_in(wid * cols, 0, cols)

        def group(gg, carry):
            for b in range(2):
                t = gg * 2 + b
                c0 = (wid + t * nw) * cols
                wait_in(b, cols)

                @pl.when(t < base_batches - 1)
                def _():
                    start_in(c0 + nw * cols, 1 - b, cols)

                @pl.when(gg >= 1)
                def _():
                    wait_out(b, cols)

                transpose(b, cols)
                start_out(c0, b, cols)
            return carry

        lax.fori_loop(0, base_batches // 2, group, 0)
        if base_batches % 2:
            t_last = base_batches - 1
            c0_last = (wid + t_last * nw) * cols
            wait_in(0, cols)
            wait_out(0, cols)
            transpose(0, cols)
            start_out(c0_last, 0, cols)
        wait_out(0, cols)
        wait_out(1, cols)

        if extra_cols:

            @pl.when(wid < extra_cols)
            def _():
                c0 = base_batches * nw * cols + wid * 128
                start_in(c0, 0, 128)
                wait_in(0, 128)
                transpose(0, 128)
                start_out(c0, 0, 128)
                wait_out(0, 128)

        if tail:

            @pl.when(wid == extra_cols)
            def _():
                buf = out_v[1].at[pl.ds(0, tail * d)]
                pltpu.sync_copy(tail_hbm, buf)
                pltpu.sync_copy(buf, r_hbm.at[pl.ds(full_blocks * 128 * d, tail * d)])

    return transpose_kernel


@functools.lru_cache(maxsize=None)
def _make_gather(bsz: int, tsz: int, d: int):
    n = bsz * tsz
    info = plsc.get_sparse_core_info()
    nw = info.num_cores * info.num_subcores
    nb_total = bsz // 128
    pairs = tsz * nb_total
    assert pairs % nw == 0
    per_w = pairs // nw
    pg = 4
    assert per_w % (2 * pg) == 0
    k_groups = per_w // pg
    chunk = pg * 128
    njb = d // 8
    mesh = plsc.VectorSubcoreMesh(core_axis_name="c", subcore_axis_name="s")

    scratch = (
        [pltpu.VMEM((per_w * 128,), jnp.int32)]
        + [pltpu.VMEM((chunk, d), jnp.float32) for _ in range(2)]
        + [pltpu.VMEM((pg * d, 129), jnp.float32) for _ in range(2)]
        + [pltpu.SemaphoreType.DMA] * 5
    )

    @functools.partial(
        pl.kernel,
        mesh=mesh,
        out_type=jax.ShapeDtypeStruct((tsz, njb, nb_total, 8, 128), jnp.float32),
        scratch_types=scratch,
        compiler_params=pltpu.CompilerParams(
            use_tc_tiling_on_sc=False, needs_layout_passes=False
        ),
    )
    def gather_kernel(idx_hbm, table_hbm, out_hbm, idx_v, r0, r1, o0, o1,
                      sem_i, sg0, sg1, ss0, ss1):
        rows_v = (r0, r1)
        out_v = (o0, o1)
        sem_g = (sg0, sg1)
        sem_s = (ss0, ss1)
        wid = lax.axis_index("s") * info.num_cores + lax.axis_index("c")
        base_pair = wid * per_w

        pltpu.sync_copy(idx_hbm.at[pl.ds(base_pair * 128, per_w * 128)], idx_v)

        iota = lax.iota(jnp.int32, 16)

        def start_gather(q, b):
            pltpu.async_copy(
                table_hbm.at[idx_v.at[pl.ds(q * chunk, chunk)]],
                rows_v[b], sem_g[b],
            )

        def wait_gather(b):
            pltpu.make_async_copy(
                table_hbm.at[idx_v.at[pl.ds(0, chunk)]], rows_v[b], sem_g[b]
            ).wait()

        def start_store(q, b):
            for p in range(pg):
                pair = base_pair + q * pg + p
                t = pair // nb_total
                nb = lax.rem(pair, nb_total)
                for jb in range(njb):
                    pltpu.async_copy(
                        out_v[b].at[pl.ds((p * njb + jb) * 8, 8), pl.ds(0, 128)],
                        out_hbm.at[t, jb, nb],
                        sem_s[b],
                    )

        def wait_store(b):
            for p in range(pg):
                for jb in range(njb):
                    pltpu.make_async_copy(
                        out_v[b].at[pl.ds((p * njb + jb) * 8, 8), pl.ds(0, 128)],
                        out_hbm.at[0, jb, 0],
                        sem_s[b],
                    ).wait()

        def transpose(b):
            for p in range(pg):
                row_lo = iota + (p * d)
                row_hi = row_lo + 16

                def body(nl, carry):
                    r = p * 128 + nl
                    nl_vec = jnp.zeros((16,), jnp.int32) + nl
                    lo = rows_v[b][r, pl.ds(0, 16)]
                    hi = rows_v[b][r, pl.ds(16, 16)]
                    plsc.store_scatter(out_v[b], [row_lo, nl_vec], lo)
                    plsc.store_scatter(out_v[b], [row_hi, nl_vec], hi)
                    return carry

                lax.fori_loop(0, 128, body, 0, unroll=4)

        start_gather(0, 0)

        def group(g, carry):
            for b in range(2):
                q = g * 2 + b

                @pl.when(q < k_groups - 1)
                def _():
                    start_gather(q + 1, 1 - b)

                wait_gather(b)

                @pl.when(g >= 1)
                def _():
                    wait_store(b)

                transpose(b)
                start_store(q, b)
            return carry

        lax.fori_loop(0, k_groups // 2, group, 0)
        wait_store(0)
        wait_store(1)

    return gather_kernel


def kernel(t_index, emb):
    b, t = t_index.shape
    v, d = emb.shape
    idx_t_flat = t_index.T.reshape(-1)
    tail = v % 128
    tail_lin = emb[v - tail:].reshape(-1)
    r1d = _make_transpose(v, d)(emb.T, tail_lin)
    ot5 = _make_gather(b, t, d)(idx_t_flat, r1d.reshape(v, d))
    return ot5.transpose(2, 4, 0, 1, 3).reshape(b, t, d)


# speedup vs baseline: 1.2048x; 1.2048x over previous
import functools

import jax
import jax.numpy as jnp
from jax import lax
from jax.experimental import pallas as pl
from jax.experimental.pallas import tpu as pltpu
from jax.experimental.pallas import tpu_sc as plsc


@functools.lru_cache(maxsize=None)
def _make_gather(bsz: int, tsz: int, d: int):
    n = bsz * tsz
    info = plsc.get_sparse_core_info()
    nw = info.num_cores * info.num_subcores
    nb_total = bsz // 128
    pairs = tsz * nb_total
    assert pairs % nw == 0
    per_w = pairs // nw
    pg = 4
    assert per_w % (2 * pg) == 0
    k_groups = per_w // pg
    chunk = pg * 128
    njb = d // 8
    mesh = plsc.VectorSubcoreMesh(core_axis_name="c", subcore_axis_name="s")

    scratch = (
        [pltpu.VMEM((per_w * 128,), jnp.int32)]
        + [pltpu.VMEM((chunk, d), jnp.float32) for _ in range(2)]
        + [pltpu.VMEM((pg * d, 129), jnp.float32) for _ in range(2)]
        + [pltpu.SemaphoreType.DMA] * 5
    )

    @functools.partial(
        pl.kernel,
        mesh=mesh,
        out_type=jax.ShapeDtypeStruct((tsz, njb, nb_total, 8, 128), jnp.float32),
        scratch_types=scratch,
        compiler_params=pltpu.CompilerParams(
            use_tc_tiling_on_sc=False, needs_layout_passes=False
        ),
    )
    def gather_kernel(idx_hbm, table_hbm, out_hbm, idx_v, r0, r1, o0, o1,
                      sem_i, sg0, sg1, ss0, ss1):
        rows_v = (r0, r1)
        out_v = (o0, o1)
        sem_g = (sg0, sg1)
        sem_s = (ss0, ss1)
        wid = lax.axis_index("s") * info.num_cores + lax.axis_index("c")
        base_pair = wid * per_w

        pltpu.sync_copy(idx_hbm.at[pl.ds(base_pair * 128, per_w * 128)], idx_v)

        iota = lax.iota(jnp.int32, 16)

        def start_gather(q, b):
            pltpu.async_copy(
                table_hbm.at[idx_v.at[pl.ds(q * chunk, chunk)]],
                rows_v[b], sem_g[b],
            )

        def wait_gather(b):
            pltpu.make_async_copy(
                table_hbm.at[idx_v.at[pl.ds(0, chunk)]], rows_v[b], sem_g[b]
            ).wait()

        def start_store(q, b):
            for p in range(pg):
                pair = base_pair + q * pg + p
                t = pair // nb_total
                nb = lax.rem(pair, nb_total)
                for jb in range(njb):
                    pltpu.async_copy(
                        out_v[b].at[pl.ds((p * njb + jb) * 8, 8), pl.ds(0, 128)],
                        out_hbm.at[t, jb, nb],
                        sem_s[b],
                    )

        def wait_store(b):
            for p in range(pg):
                for jb in range(njb):
                    pltpu.make_async_copy(
                        out_v[b].at[pl.ds((p * njb + jb) * 8, 8), pl.ds(0, 128)],
                        out_hbm.at[0, jb, 0],
                        sem_s[b],
                    ).wait()

        def transpose(b):
            for p in range(pg):
                row_lo = iota + (p * d)
                row_hi = row_lo + 16

                def body(nl, carry):
                    r = p * 128 + nl
                    nl_vec = jnp.zeros((16,), jnp.int32) + nl
                    lo = rows_v[b][r, pl.ds(0, 16)]
                    hi = rows_v[b][r, pl.ds(16, 16)]
                    plsc.store_scatter(out_v[b], [row_lo, nl_vec], lo)
                    plsc.store_scatter(out_v[b], [row_hi, nl_vec], hi)
                    return carry

                lax.fori_loop(0, 128, body, 0, unroll=4)

        start_gather(0, 0)

        def group(g, carry):
            for b in range(2):
                q = g * 2 + b

                @pl.when(q < k_groups - 1)
                def _():
                    start_gather(q + 1, 1 - b)

                wait_gather(b)

                @pl.when(g >= 1)
                def _():
                    wait_store(b)

                transpose(b)
                start_store(q, b)
            return carry

        lax.fori_loop(0, k_groups // 2, group, 0)
        wait_store(0)
        wait_store(1)

    return gather_kernel


def kernel(t_index, emb):
    b, t = t_index.shape
    v, d = emb.shape
    idx_t_flat = t_index.T.reshape(-1)
    ot5 = _make_gather(b, t, d)(idx_t_flat, emb)
    return ot5.transpose(2, 4, 0, 1, 3).reshape(b, t, d)
